# baseline (speedup 1.0000x reference)
import functools

import jax
import jax.numpy as jnp
from jax import lax
from jax.experimental import pallas as pl
from jax.experimental.pallas import tpu as pltpu
from jax.experimental.pallas import tpu_sc as plsc

L = 16
STAGES = 15


def _build(B, total, maxlen):
    info = plsc.get_sparse_core_info()
    NW = info.num_cores * info.num_subcores
    LO = (B // NW) // L * L
    REM = (B - LO * NW) // L
    HI = LO + (L if REM else 0)
    CH = 2 * L * (-(-HI // (STAGES * 2 * L)))
    NGRP = CH // L

    CUSZ = CH + 8
    VSZ_BASE = maxlen * CH + 24
    VSZ = VSZ_BASE + ((total - VSZ_BASE) % 8)
    V_CLAMP = total - VSZ
    NSUB = 4
    SUB = -(-VSZ // (NSUB * 8)) * 8
    SUBL = VSZ - (NSUB - 1) * SUB

    mesh = plsc.VectorSubcoreMesh(core_axis_name="c", subcore_axis_name="s")

    @functools.partial(
        pl.kernel,
        mesh=mesh,
        out_type=jax.ShapeDtypeStruct((B,), jnp.float32),
        compiler_params=pltpu.CompilerParams(needs_layout_passes=False),
        scratch_types=[
            pltpu.VMEM((VSZ,), jnp.float32),
            pltpu.VMEM((VSZ,), jnp.float32),
            pltpu.VMEM((CUSZ,), jnp.int32),
            pltpu.VMEM((CUSZ,), jnp.int32),
            pltpu.VMEM((CUSZ,), jnp.int32),
            pltpu.VMEM((CH,), jnp.float32),
            pltpu.VMEM((CH,), jnp.float32),
            pltpu.VMEM((maxlen + 1, L), jnp.float32),
            pltpu.SemaphoreType.DMA,
            pltpu.SemaphoreType.DMA,
            pltpu.SemaphoreType.DMA,
            pltpu.SemaphoreType.DMA,
            pltpu.SemaphoreType.DMA,
            pltpu.SemaphoreType.DMA,
            pltpu.SemaphoreType.DMA,
        ],
    )
    def sck(vals_hbm, cu_hbm, waux_hbm, out_hbm,
            valv0, valv1, cuv0, cuv1, cuv2, outv0, outv1, wv,
            sv0, sv1, sc0, sc1, sc2, so0, so1):
        wid = lax.axis_index("s") * info.num_cores + lax.axis_index("c")
        pltpu.sync_copy(waux_hbm, wv)
        wrows = [wv[j] for j in range(maxlen)]
        bv = wv[maxlen]

        valvs, vsems = [valv0, valv1], [sv0, sv1]
        cuvs, csems = [cuv0, cuv1, cuv2], [sc0, sc1, sc2]
        outvs, osems = [outv0, outv1], [so0, so1]

        row0 = LO * wid + L * jnp.maximum(wid - (NW - REM), 0)
        rows_w = LO + L * (wid >= NW - REM).astype(jnp.int32)

        starts, vbases, needs = {}, {}, {}

        def issue_cu(k):
            start = row0 + jnp.minimum(k * CH, rows_w - CH)
            starts[k] = start
            return pltpu.async_copy(
                cu_hbm.at[pl.ds(start, CH + 1)],
                cuvs[k % 3].at[pl.ds(0, CH + 1)], csems[k % 3])

        def issue_val(k):
            cuv = cuvs[k % 3]
            vstart = cuv[pl.ds(0, L)][0]
            vend = cuv[pl.ds(CH - 15, L)][15]
            vbase = jnp.minimum((vstart // 8) * 8, V_CLAMP)
            vbases[k] = vbase
            needs[k] = need = vend - vbase
            valv, sem = valvs[k % 2], vsems[k % 2]
            pltpu.async_copy(vals_hbm.at[pl.ds(vbase, SUB)],
                             valv.at[pl.ds(0, SUB)], sem)
            for i in range(1, NSUB):
                sz = SUB if i < NSUB - 1 else SUBL
                @pl.when(need > i * SUB)
                def _(i=i, sz=sz):
                    pltpu.async_copy(
                        vals_hbm.at[pl.ds(vbase + i * SUB, sz)],
                        valv.at[pl.ds(i * SUB, sz)], sem)

        def wait_val(k):
            need = needs[k]
            valv, sem = valvs[k % 2], vsems[k % 2]
            pltpu.make_async_copy(vals_hbm.at[pl.ds(0, SUB)],
                                  valv.at[pl.ds(0, SUB)], sem).wait()
            for i in range(1, NSUB):
                sz = SUB if i < NSUB - 1 else SUBL
                @pl.when(need > i * SUB)
                def _(i=i, sz=sz):
                    pltpu.make_async_copy(
                        vals_hbm.at[pl.ds(0, sz)],
                        valv.at[pl.ds(i * SUB, sz)], sem).wait()

        def compute(k):
            cuv, valv, outv = cuvs[k % 3], valvs[k % 2], outvs[k % 2]
            vbase = vbases[k]

            @plsc.parallel_loop(0, NGRP, 1, unroll=4)
            def group(g):
                off = g * L
                cur = cuv[pl.ds(off, L)]
                nxt = cuv[pl.ds(off + 1, L)]
                ln = nxt - cur
                rel = cur - vbase
                acc = bv
                for j in range(maxlen):
                    m = ln > j
                    gj = plsc.load_gather(valv, [rel + j], mask=m)
                    acc = acc + jnp.where(m, gj * wrows[j], 0.0)
                outv[pl.ds(off, L)] = acc
            return pltpu.async_copy(
                outv, out_hbm.at[pl.ds(starts[k], CH)], osems[k % 2])

        h_cu, h_out = {}, [None, None]
        h_cu[0] = issue_cu(0)
        h_cu[0].wait()
        issue_val(0)
        if STAGES > 1:
            h_cu[1] = issue_cu(1)
        for k in range(STAGES):
            if k + 1 < STAGES:
                h_cu[k + 1].wait()
                issue_val(k + 1)
            if k + 2 < STAGES:
                h_cu[k + 2] = issue_cu(k + 2)
            wait_val(k)
            if h_out[k % 2] is not None:
                h_out[k % 2].wait()
            h_out[k % 2] = compute(k)
        for h in h_out:
            if h is not None:
                h.wait()

    return sck


def kernel(values, cu_seqlens, seqlens, W, b):
    B = cu_seqlens.shape[0] - 1
    maxlen = W.shape[0]
    total = values.shape[0]
    min_total = maxlen * (B // (2 * STAGES) + 256) + 256
    if total < min_total:
        values = jnp.pad(values, (0, min_total - total))
        total = min_total
    waux = jnp.concatenate(
        [
            jnp.broadcast_to(W.reshape(maxlen, 1), (maxlen, L)),
            jnp.broadcast_to(b.reshape(1, 1), (1, L)),
        ],
        axis=0,
    ).astype(jnp.float32)
    out = _build(B, total, maxlen)(values, cu_seqlens, waux)
    return out.reshape(B, 1)


# speedup vs baseline: 1.0302x; 1.0302x over previous
import functools

import jax
import jax.numpy as jnp
from jax import lax
from jax.experimental import pallas as pl
from jax.experimental.pallas import tpu as pltpu
from jax.experimental.pallas import tpu_sc as plsc

L = 16
STAGES = 15


def _build(B, total, maxlen):
    info = plsc.get_sparse_core_info()
    NW = info.num_cores * info.num_subcores
    LO = (B // NW) // L * L
    REM = (B - LO * NW) // L
    HI = LO + (L if REM else 0)
    CH = 2 * L * (-(-HI // (STAGES * 2 * L)))
    NGRP = CH // L

    CUSZ = CH + 8
    VSZ_BASE = maxlen * CH + 24
    VSZ = VSZ_BASE + ((total - VSZ_BASE) % 8)
    V_CLAMP = total - VSZ

    mesh = plsc.VectorSubcoreMesh(core_axis_name="c", subcore_axis_name="s")

    @functools.partial(
        pl.kernel,
        mesh=mesh,
        out_type=jax.ShapeDtypeStruct((B,), jnp.float32),
        compiler_params=pltpu.CompilerParams(needs_layout_passes=False),
        scratch_types=[
            pltpu.VMEM((VSZ,), jnp.float32),
            pltpu.VMEM((VSZ,), jnp.float32),
            pltpu.VMEM((CUSZ,), jnp.int32),
            pltpu.VMEM((CUSZ,), jnp.int32),
            pltpu.VMEM((CUSZ,), jnp.int32),
            pltpu.VMEM((CH,), jnp.float32),
            pltpu.VMEM((CH,), jnp.float32),
            pltpu.VMEM((maxlen + 1, L), jnp.float32),
            pltpu.SemaphoreType.DMA,
            pltpu.SemaphoreType.DMA,
            pltpu.SemaphoreType.DMA,
            pltpu.SemaphoreType.DMA,
            pltpu.SemaphoreType.DMA,
            pltpu.SemaphoreType.DMA,
            pltpu.SemaphoreType.DMA,
        ],
    )
    def sck(vals_hbm, cu_hbm, waux_hbm, out_hbm,
            valv0, valv1, cuv0, cuv1, cuv2, outv0, outv1, wv,
            sv0, sv1, sc0, sc1, sc2, so0, so1):
        wid = lax.axis_index("s") * info.num_cores + lax.axis_index("c")
        pltpu.sync_copy(waux_hbm, wv)
        wrows = [wv[j] for j in range(maxlen)]
        bv = wv[maxlen]

        valvs, vsems = [valv0, valv1], [sv0, sv1]
        cuvs, csems = [cuv0, cuv1, cuv2], [sc0, sc1, sc2]
        outvs, osems = [outv0, outv1], [so0, so1]

        row0 = LO * wid + L * jnp.maximum(wid - (NW - REM), 0)
        rows_w = LO + L * (wid >= NW - REM).astype(jnp.int32)

        starts, vbases = {}, {}

        def issue_cu(k):
            start = row0 + jnp.minimum(k * CH, rows_w - CH)
            starts[k] = start
            return pltpu.async_copy(
                cu_hbm.at[pl.ds(start, CH + 1)],
                cuvs[k % 3].at[pl.ds(0, CH + 1)], csems[k % 3])

        def issue_val(k):
            vstart = cuvs[k % 3][pl.ds(0, L)][0]
            vbase = jnp.minimum((vstart // 8) * 8, V_CLAMP)
            vbases[k] = vbase
            return pltpu.async_copy(
                vals_hbm.at[pl.ds(vbase, VSZ)], valvs[k % 2], vsems[k % 2])

        def compute(k):
            cuv, valv, outv = cuvs[k % 3], valvs[k % 2], outvs[k % 2]
            vbase = vbases[k]

            @plsc.parallel_loop(0, NGRP, 1, unroll=4)
            def group(g):
                off = g * L
                cur = cuv[pl.ds(off, L)]
                nxt = cuv[pl.ds(off + 1, L)]
                ln = nxt - cur
                rel = cur - vbase
                acc = bv
                for j in range(maxlen):
                    m = ln > j
                    gj = plsc.load_gather(valv, [rel + j], mask=m)
                    acc = acc + jnp.where(m, gj * wrows[j], 0.0)
                outv[pl.ds(off, L)] = acc
            return pltpu.async_copy(
                outv, out_hbm.at[pl.ds(starts[k], CH)], osems[k % 2])

        h_cu, h_val, h_out = {}, {}, [None, None]
        h_cu[0] = issue_cu(0)
        h_cu[0].wait()
        h_val[0] = issue_val(0)
        if STAGES > 1:
            h_cu[1] = issue_cu(1)
        for k in range(STAGES):
            if k + 1 < STAGES:
                h_cu[k + 1].wait()
                h_val[k + 1] = issue_val(k + 1)
            if k + 2 < STAGES:
                h_cu[k + 2] = issue_cu(k + 2)
            h_val[k].wait()
            if h_out[k % 2] is not None:
                h_out[k % 2].wait()
            h_out[k % 2] = compute(k)
        for h in h_out:
            if h is not None:
                h.wait()

    return sck


def kernel(values, cu_seqlens, seqlens, W, b):
    B = cu_seqlens.shape[0] - 1
    maxlen = W.shape[0]
    total = values.shape[0]
    min_total = maxlen * (B // (2 * STAGES) + 256) + 256
    if total < min_total:
        values = jnp.pad(values, (0, min_total - total))
        total = min_total
    waux = jnp.concatenate(
        [
            jnp.broadcast_to(W.reshape(maxlen, 1), (maxlen, L)),
            jnp.broadcast_to(b.reshape(1, 1), (1, L)),
        ],
        axis=0,
    ).astype(jnp.float32)
    out = _build(B, total, maxlen)(values, cu_seqlens, waux)
    return out.reshape(B, 1)
